# Initial kernel scaffold; baseline (speedup 1.0000x reference)
import jax
import jax.numpy as jnp
from jax.experimental import pallas as pl


def kernel(feature_idx, feature_vals, W, b):
    raise NotImplementedError("write your pallas kernel here")


# speedup vs baseline: 1.7224x; 1.7224x over previous
import jax
import jax.numpy as jnp
from jax import lax
from jax.experimental import pallas as pl
from jax.experimental.pallas import tpu as pltpu
from jax.experimental.pallas import tpu_sc as plsc

B, F, V = 16384, 26, 1000000
L = 16
NC, NS = 2, 16
NW = NC * NS
ROWS_W = B // NW
GCH = 128
CPW = ROWS_W // GCH
FIRE = 8
NGATHER = F * CPW
GROUPS = NGATHER // FIRE


def _sc_body(idx_hbm, vals_hbm, w_hbm, b_hbm, out_hbm,
             idx_v, v_v, g_v, b_v, out_v, sem):
    wid = lax.axis_index("s") * NC + lax.axis_index("c")
    col0 = wid * ROWS_W

    pltpu.sync_copy(idx_hbm.at[:, pl.ds(wid * CPW, CPW), :], idx_v)
    pltpu.sync_copy(vals_hbm.at[:, pl.ds(col0, ROWS_W)], v_v)
    pltpu.sync_copy(b_hbm, b_v)

    def gather_group(t, carry):
        cps = []
        for j in range(FIRE):
            r = t * FIRE + j
            f = r // CPW
            c = r % CPW
            cps.append(pltpu.async_copy(
                w_hbm.at[idx_v.at[f, c]],
                g_v.at[f, pl.ds(c * GCH, GCH)], sem))
        for cp in cps:
            cp.wait()
        return carry
    lax.fori_loop(0, GROUPS, gather_group, 0)

    bvec = b_v[...]

    def colgroup(sg, carry):
        s = pl.ds(sg * L, L)
        acc = bvec
        for f in range(F):
            acc = acc + g_v[f, s] * v_v[f, s]
        out_v[s] = 1.0 / (1.0 + jnp.exp(-acc))
        return carry
    lax.fori_loop(0, ROWS_W // L, colgroup, 0)

    pltpu.sync_copy(out_v, out_hbm.at[pl.ds(col0, ROWS_W)])


def kernel(feature_idx, feature_vals, W, b):
    idx_t = feature_idx.astype(jnp.int32).T.reshape(F, B // GCH, GCH)
    vals_t = feature_vals.T
    b16 = jnp.broadcast_to(jnp.asarray(b, jnp.float32).reshape(()), (L,))

    mesh = plsc.VectorSubcoreMesh(core_axis_name="c", subcore_axis_name="s")
    run = pl.kernel(
        _sc_body,
        out_type=jax.ShapeDtypeStruct((B,), jnp.float32),
        mesh=mesh,
        scratch_types=[
            pltpu.VMEM((F, CPW, GCH), jnp.int32),
            pltpu.VMEM((F, ROWS_W), jnp.float32),
            pltpu.VMEM((F, ROWS_W), jnp.float32),
            pltpu.VMEM((L,), jnp.float32),
            pltpu.VMEM((ROWS_W,), jnp.float32),
            pltpu.SemaphoreType.DMA,
        ],
    )
    return run(idx_t, vals_t, W, b16)


# speedup vs baseline: 1.8673x; 1.0841x over previous
import jax
import jax.numpy as jnp
from jax import lax
from jax.experimental import pallas as pl
from jax.experimental.pallas import tpu as pltpu
from jax.experimental.pallas import tpu_sc as plsc

B, F, V = 16384, 26, 1000000
L = 16
NC, NS = 2, 16
NW = NC * NS
ROWS_W = B // NW
E = F * ROWS_W


def _sc_body(idx_hbm, vals_hbm, w_hbm, b_hbm, out_hbm,
             idx_v, v_v, g_v, b_v, out_v, sem):
    wid = lax.axis_index("s") * NC + lax.axis_index("c")
    base = wid * E

    pltpu.sync_copy(idx_hbm.at[pl.ds(base, E)], idx_v)
    gcp = pltpu.async_copy(w_hbm.at[idx_v], g_v, sem)
    pltpu.sync_copy(vals_hbm.at[pl.ds(base, E)], v_v)
    pltpu.sync_copy(b_hbm, b_v)
    gcp.wait()

    bvec = b_v[...]

    def colgroup(sg, carry):
        acc = bvec
        for f in range(F):
            s = pl.ds(f * ROWS_W + sg * L, L)
            acc = acc + g_v[s] * v_v[s]
        out_v[pl.ds(sg * L, L)] = 1.0 / (1.0 + jnp.exp(-acc))
        return carry
    lax.fori_loop(0, ROWS_W // L, colgroup, 0)

    pltpu.sync_copy(out_v, out_hbm.at[pl.ds(wid * ROWS_W, ROWS_W)])


def kernel(feature_idx, feature_vals, W, b):
    idx_bt = (feature_idx.astype(jnp.int32)
              .reshape(NW, ROWS_W, F).transpose(0, 2, 1).reshape(NW * E))
    vals_bt = feature_vals.reshape(NW, ROWS_W, F).transpose(0, 2, 1).reshape(NW * E)
    b16 = jnp.broadcast_to(jnp.asarray(b, jnp.float32).reshape(()), (L,))

    mesh = plsc.VectorSubcoreMesh(core_axis_name="c", subcore_axis_name="s")
    run = pl.kernel(
        _sc_body,
        out_type=jax.ShapeDtypeStruct((B,), jnp.float32),
        mesh=mesh,
        scratch_types=[
            pltpu.VMEM((E,), jnp.int32),
            pltpu.VMEM((E,), jnp.float32),
            pltpu.VMEM((E,), jnp.float32),
            pltpu.VMEM((L,), jnp.float32),
            pltpu.VMEM((ROWS_W,), jnp.float32),
            pltpu.SemaphoreType.DMA,
        ],
    )
    return run(idx_bt, vals_bt, W, b16)
